# Initial kernel scaffold; baseline (speedup 1.0000x reference)
import jax
import jax.numpy as jnp
from jax.experimental import pallas as pl


def kernel(x, gamma, beta):
    raise NotImplementedError("write your pallas kernel here")


# speedup vs baseline: 2.2666x; 2.2666x over previous
import functools

import jax
import jax.numpy as jnp
from jax.experimental import pallas as pl
from jax.experimental.pallas import tpu as pltpu

_EPS = 1e-5


def _bn_relu_kernel(x_ref, g_ref, b_ref, o_ref, *, inv_count):
    x = x_ref[...].astype(jnp.float32)
    mean = jnp.sum(x, axis=(0, 2), keepdims=True) * inv_count
    d = x - mean
    var = jnp.sum(d * d, axis=(0, 2), keepdims=True) * inv_count
    scale = jax.lax.rsqrt(var + _EPS) * g_ref[...].reshape(1, -1, 1)
    y = d * scale + b_ref[...].reshape(1, -1, 1)
    o_ref[...] = jnp.maximum(y, 0.0).astype(o_ref.dtype)


def _cost(flops, bytes_accessed):
    try:
        return pl.CostEstimate(flops=int(flops), transcendentals=0,
                               bytes_accessed=int(bytes_accessed))
    except Exception:
        return None


@jax.jit
def _bn_relu(x, gamma, beta):
    N, C, T, V = x.shape
    M = T * V
    itemsize = jnp.dtype(x.dtype).itemsize

    budget = 40 << 20
    cb = C
    while cb > 1 and (4 * N * cb * M * itemsize > budget or C % cb != 0):
        cb //= 2

    x3 = x.reshape(N, C, M)
    y3 = pl.pallas_call(
        functools.partial(_bn_relu_kernel, inv_count=1.0 / (N * M)),
        out_shape=jax.ShapeDtypeStruct((N, C, M), x.dtype),
        grid=(C // cb,),
        in_specs=[
            pl.BlockSpec((N, cb, M), lambda c: (0, c, 0)),
            pl.BlockSpec((cb, 1), lambda c: (c, 0)),
            pl.BlockSpec((cb, 1), lambda c: (c, 0)),
        ],
        out_specs=pl.BlockSpec((N, cb, M), lambda c: (0, c, 0)),
        compiler_params=pltpu.CompilerParams(
            dimension_semantics=("parallel",),
            vmem_limit_bytes=64 << 20),
        cost_estimate=_cost(6 * N * C * M,
                            2 * N * C * M * itemsize + 2 * C * 4),
    )(x3,
      gamma.astype(jnp.float32).reshape(C, 1),
      beta.astype(jnp.float32).reshape(C, 1))
    return y3.reshape(N, C, T, V)


def kernel(x, gamma, beta):
    return _bn_relu(x, gamma, beta), 0
